# Initial kernel scaffold; baseline (speedup 1.0000x reference)
import jax
import jax.numpy as jnp
from jax.experimental import pallas as pl


def kernel(x, edge_index, W_enc, b_enc, W1, a_src1, a_dst1, b1, g1_w, g1_b, g1_ms, W2, a_src2, a_dst2, b2, g2_w, g2_b, g2_ms, W_fc, b_fc):
    raise NotImplementedError("write your pallas kernel here")


# speedup vs baseline: 31.1061x; 31.1061x over previous
import functools

import jax
import jax.numpy as jnp
from jax import lax
from jax.experimental import pallas as pl
from jax.experimental.pallas import tpu as pltpu
from jax.experimental.pallas import tpu_sc as plsc

_N = 100000
_E = 1600000
_NC, _NS = 2, 16
_NW = _NC * _NS
_E2 = 1638400
_R2 = _E2 // 128
_RREAL = _E // 128
_RPW = _R2 // _NW
_RPT = _R2 // _NS
_N2 = 100096
_NPT = _N2 // _NS
_BN = 2000
_G = _N // _BN

_f32 = jnp.float32
_i32 = jnp.int32


def _sc_mesh():
    return plsc.VectorSubcoreMesh(core_axis_name="c", subcore_axis_name="s",
                                  num_cores=_NC, num_subcores=_NS)


def _p1_gather(src2d, tab):
    @functools.partial(
        pl.kernel,
        out_type=jax.ShapeDtypeStruct((_R2, 128), _f32),
        mesh=_sc_mesh(),
        compiler_params=pltpu.CompilerParams(needs_layout_passes=False, use_tc_tiling_on_sc=False),
        scratch_types=[
            pltpu.VMEM((_N,), _f32),
            pltpu.VMEM((16, 128), _i32),
            pltpu.VMEM((16, 128), _f32),
        ],
    )
    def k(src_h, tab_h, u_h, tab_v, idx_v, u_v):
        cid = lax.axis_index("c")
        sid = lax.axis_index("s")
        wid = sid * _NC + cid
        pltpu.sync_copy(tab_h, tab_v)
        rbase = wid * _RPW
        fill = jnp.full((16,), -1e30, _f32)

        def chunk(ci, _):
            roff = rbase + ci * 16
            pltpu.sync_copy(src_h.at[pl.ds(roff, 16)], idx_v)

            def grp(g, _):
                kk = g // 8
                ii = g % 8
                idx = idx_v[kk, pl.ds(ii * 16, 16)]
                vals = plsc.load_gather(tab_v, [idx])
                real = (roff + kk) < _RREAL
                u_v[kk, pl.ds(ii * 16, 16)] = jnp.where(real, vals, fill)
                return 0

            lax.fori_loop(0, 128, grp, 0)
            pltpu.sync_copy(u_v, u_h.at[pl.ds(roff, 16)])
            return 0

        lax.fori_loop(0, _RPW // 16, chunk, 0)

    return k(src2d, tab)


def _p2_softmax_num(dst2d, u2d, tab, mhat16):
    @functools.partial(
        pl.kernel,
        out_type=(
            jax.ShapeDtypeStruct((_R2, 128), _f32),
            jax.ShapeDtypeStruct((_NC * _N2,), _f32),
        ),
        mesh=_sc_mesh(),
        compiler_params=pltpu.CompilerParams(needs_layout_passes=False, use_tc_tiling_on_sc=False),
        scratch_types=[
            pltpu.VMEM((_N,), _f32),
            pltpu.VMEM((16, 128), _i32),
            pltpu.VMEM((16, 128), _f32),
            pltpu.VMEM((16, 128), _f32),
            pltpu.VMEM((16,), _f32),
            pltpu.VMEM((_NPT,), _f32),
            pltpu.VMEM_SHARED((_N2,), _f32),
        ],
    )
    def k(dst_h, u_h, tab_h, mh_h, e_h, sp_h,
          tab_v, didx_v, u_v, e_v, mh_v, zb_v, s_sh):
        cid = lax.axis_index("c")
        sid = lax.axis_index("s")
        wid = sid * _NC + cid
        pltpu.sync_copy(tab_h, tab_v)
        pltpu.sync_copy(mh_h, mh_v)

        def zi(i, _):
            zb_v[pl.ds(i * 16, 16)] = jnp.zeros((16,), _f32)
            return 0

        lax.fori_loop(0, _NPT // 16, zi, 0)
        pltpu.sync_copy(zb_v, s_sh.at[pl.ds(sid * _NPT, _NPT)])
        plsc.subcore_barrier()
        mh = mh_v[...]
        rbase = wid * _RPW

        def chunk(ci, _):
            roff = rbase + ci * 16
            pltpu.sync_copy(dst_h.at[pl.ds(roff, 16)], didx_v)
            pltpu.sync_copy(u_h.at[pl.ds(roff, 16)], u_v)

            def grp(g, _):
                kk = g // 8
                ii = g % 8
                idx = didx_v[kk, pl.ds(ii * 16, 16)]
                v = plsc.load_gather(tab_v, [idx])
                a = u_v[kk, pl.ds(ii * 16, 16)] + v
                a = jnp.where(a > 0, a, 0.2 * a)
                e_v[kk, pl.ds(ii * 16, 16)] = jnp.exp(a - mh)
                return 0

            lax.fori_loop(0, 128, grp, 0)
            pltpu.sync_copy(e_v, e_h.at[pl.ds(roff, 16)])

            def sca(kk, _):
                pltpu.sync_copy(e_v.at[kk], s_sh.at[didx_v.at[kk]], add=True)
                return 0

            lax.fori_loop(0, 16, sca, 0)
            return 0

        lax.fori_loop(0, _RPW // 16, chunk, 0)
        plsc.subcore_barrier()
        pltpu.sync_copy(s_sh.at[pl.ds(sid * _NPT, _NPT)], zb_v)
        pltpu.sync_copy(zb_v, sp_h.at[pl.ds(cid * _N2 + sid * _NPT, _NPT)])

    return k(dst2d, u2d, tab, mhat16)


def _pb_aggregate(src2d, dst2d, e2d, ha, hb):
    @functools.partial(
        pl.kernel,
        out_type=jax.ShapeDtypeStruct((_NC, _N2, 16), _f32),
        mesh=_sc_mesh(),
        compiler_params=pltpu.CompilerParams(needs_layout_passes=False, use_tc_tiling_on_sc=False),
        scratch_types=[
            pltpu.VMEM((16, 128), _i32),
            pltpu.VMEM((16, 128), _i32),
            pltpu.VMEM((16, 128), _f32),
            pltpu.VMEM((128, 16), _f32),
            pltpu.VMEM((368, 16), _f32),
            pltpu.VMEM_SHARED((_N2, 16), _f32),
            pltpu.SemaphoreType.DMA,
        ],
    )
    def k(src_h, dst_h, e_h, ha_h, hb_h, acc_h,
          sidx_v, didx_v, e_v, rows_v, zb_v, acc_sh, sem):
        cid = lax.axis_index("c")
        sid = lax.axis_index("s")

        def zi(g, _):
            zb_v[g] = jnp.zeros((16,), _f32)
            return 0

        lax.fori_loop(0, 368, zi, 0)

        def zc(p, _):
            pltpu.sync_copy(zb_v, acc_sh.at[pl.ds(sid * _NPT + p * 368, 368)])
            return 0

        lax.fori_loop(0, _NPT // 368, zc, 0)
        plsc.subcore_barrier()
        rbase = sid * _RPT

        def run(tab_h):
            def chunk(ci, _):
                roff = rbase + ci * 16
                pltpu.sync_copy(src_h.at[pl.ds(roff, 16)], sidx_v)
                pltpu.sync_copy(dst_h.at[pl.ds(roff, 16)], didx_v)
                pltpu.sync_copy(e_h.at[pl.ds(roff, 16)], e_v)

                def sub(kk, _):
                    pltpu.async_copy(tab_h.at[sidx_v.at[kk]], rows_v, sem).wait()

                    def scl(g, _):
                        ev = e_v[kk, pl.ds(g * 16, 16)]
                        base = g * 16
                        for j in range(16):
                            rows_v[base + j] = rows_v[base + j] * ev[j]
                        return 0

                    lax.fori_loop(0, 8, scl, 0)
                    pltpu.sync_copy(rows_v, acc_sh.at[didx_v.at[kk]], add=True)
                    return 0

                lax.fori_loop(0, 16, sub, 0)
                return 0

            lax.fori_loop(0, _RPT // 16, chunk, 0)

        @pl.when(cid == 0)
        def _():
            run(ha_h)

        @pl.when(cid == 1)
        def _():
            run(hb_h)

        plsc.subcore_barrier()

        def wc(p, _):
            off = sid * _NPT + p * 368
            pltpu.sync_copy(acc_sh.at[pl.ds(off, 368)], zb_v)
            pltpu.sync_copy(zb_v, acc_h.at[cid, pl.ds(off, 368)])
            return 0

        lax.fori_loop(0, _NPT // 368, wc, 0)

    return k(src2d, dst2d, e2d, ha, hb)


def _prep_call(act, W, a_src, a_dst, fold_params, din):
    folded = fold_params is not None

    def body(*refs):
        if folded:
            (act_r, pm1_r, pm2_r, gw_r, gb_r, gms_r, w_r, asr_r, adr_r,
             hp_r, as_r, ad_r, mh_r, ms_s, md_s) = refs
        else:
            (act_r, w_r, asr_r, adr_r,
             hp_r, as_r, ad_r, mh_r, ms_s, md_s) = refs
        i = pl.program_id(0)
        z = act_r[...]
        if folded:
            m1 = pm1_r[...] / _N
            m2 = pm2_r[...] / _N
            gms = gms_r[...]
            var = m2 - 2.0 * gms * m1 * m1 + gms * gms * m1 * m1
            inv = lax.rsqrt(var + 1e-5)
            alpha = gw_r[...] * inv
            beta = gb_r[...] - gw_r[...] * gms * m1 * inv
            z = z * alpha + beta
        h = jnp.dot(z, w_r[...], preferred_element_type=_f32)
        hp_r[0] = h[:, :16]
        hp_r[1] = h[:, 16:]
        a_s = jnp.sum(h * asr_r[...], axis=1, keepdims=True)
        a_d = jnp.sum(h * adr_r[...], axis=1, keepdims=True)
        as_r[...] = a_s
        ad_r[...] = a_d
        bm_s = jnp.max(a_s)
        bm_d = jnp.max(a_d)

        @pl.when(i == 0)
        def _():
            ms_s[0, 0] = bm_s
            md_s[0, 0] = bm_d

        @pl.when(i > 0)
        def _():
            ms_s[0, 0] = jnp.maximum(ms_s[0, 0], bm_s)
            md_s[0, 0] = jnp.maximum(md_s[0, 0], bm_d)

        @pl.when(i == _G - 1)
        def _():
            m = ms_s[0, 0] + md_s[0, 0]
            mh_r[...] = jnp.full((1, 16), jnp.where(m > 0, m, 0.2 * m), _f32)

    full = lambda shape: pl.BlockSpec(shape, lambda i: tuple(0 for _ in shape))
    in_specs = [pl.BlockSpec((_BN, din), lambda i: (i, 0))]
    args = [act]
    if folded:
        pm1, pm2, gw, gb, gms = fold_params
        in_specs += [full((1, 32))] * 5
        args += [pm1, pm2, gw, gb, gms]
    in_specs += [full((din, 32)), full((1, 32)), full((1, 32))]
    args += [W, a_src, a_dst]

    return pl.pallas_call(
        body,
        grid=(_G,),
        in_specs=in_specs,
        out_specs=[
            pl.BlockSpec((2, _BN, 16), lambda i: (0, i, 0)),
            pl.BlockSpec((_BN, 1), lambda i: (i, 0)),
            pl.BlockSpec((_BN, 1), lambda i: (i, 0)),
            pl.BlockSpec((1, 16), lambda i: (0, 0)),
        ],
        out_shape=[
            jax.ShapeDtypeStruct((2, _N, 16), _f32),
            jax.ShapeDtypeStruct((_N, 1), _f32),
            jax.ShapeDtypeStruct((_N, 1), _f32),
            jax.ShapeDtypeStruct((1, 16), _f32),
        ],
        scratch_shapes=[pltpu.SMEM((1, 1), _f32), pltpu.SMEM((1, 1), _f32)],
    )(*args)


def _enc_call(x, W_enc, b_enc):
    def body(x_r, we_r, be_r, h0_r):
        h0_r[...] = jnp.dot(x_r[...], we_r[...],
                            preferred_element_type=_f32) + be_r[...]

    return pl.pallas_call(
        body,
        grid=(_G,),
        in_specs=[
            pl.BlockSpec((_BN, 128), lambda i: (i, 0)),
            pl.BlockSpec((128, 64), lambda i: (0, 0)),
            pl.BlockSpec((1, 64), lambda i: (0, 0)),
        ],
        out_specs=pl.BlockSpec((_BN, 64), lambda i: (i, 0)),
        out_shape=jax.ShapeDtypeStruct((_N, 64), _f32),
    )(x, W_enc, b_enc)


def _post_call(acc0, acc1, s0, s1, a_s, a_d, mhat, ha, hb, b):
    def body(acc0_r, acc1_r, s0_r, s1_r, as_r, ad_r, mh_r, ha_r, hb_r, b_r,
             act_r, pm1_r, pm2_r):
        i = pl.program_id(0)
        mh0 = mh_r[0, 0]
        araw = as_r[...] + ad_r[...]
        alr = jnp.where(araw > 0, araw, 0.2 * araw)
        es = jnp.exp(alr - mh0)
        s = s0_r[...] + s1_r[...] + es
        acc = jnp.concatenate([acc0_r[...], acc1_r[...]], axis=1)
        hf = jnp.concatenate([ha_r[...], hb_r[...]], axis=1)
        acc = acc + es * hf
        o = acc / (s + 1e-16) + b_r[...]
        act = jnp.where(o > 0, o, jnp.exp(o) - 1.0)
        act_r[...] = act
        ps1 = jnp.sum(act, axis=0, keepdims=True)
        ps2 = jnp.sum(act * act, axis=0, keepdims=True)

        @pl.when(i == 0)
        def _():
            pm1_r[...] = ps1
            pm2_r[...] = ps2

        @pl.when(i > 0)
        def _():
            pm1_r[...] = pm1_r[...] + ps1
            pm2_r[...] = pm2_r[...] + ps2

    blk = lambda w: pl.BlockSpec((_BN, w), lambda i: (i, 0))
    full = lambda shape: pl.BlockSpec(shape, lambda i: tuple(0 for _ in shape))
    return pl.pallas_call(
        body,
        grid=(_G,),
        in_specs=[blk(16), blk(16), blk(1), blk(1), blk(1), blk(1),
                  full((1, 16)), blk(16), blk(16), full((1, 32))],
        out_specs=[
            pl.BlockSpec((_BN, 32), lambda i: (i, 0)),
            pl.BlockSpec((1, 32), lambda i: (0, 0)),
            pl.BlockSpec((1, 32), lambda i: (0, 0)),
        ],
        out_shape=[
            jax.ShapeDtypeStruct((_N, 32), _f32),
            jax.ShapeDtypeStruct((1, 32), _f32),
            jax.ShapeDtypeStruct((1, 32), _f32),
        ],
    )(acc0, acc1, s0, s1, a_s, a_d, mhat, ha, hb, b)


def _final_call(act, pm1, pm2, gw, gb, gms, W_fc, b_fc):
    def body(act_r, pm1_r, pm2_r, gw_r, gb_r, gms_r, wfc_r, bfc_r,
             out_r, mx_s):
        i = pl.program_id(0)
        m1 = pm1_r[...] / _N
        m2 = pm2_r[...] / _N
        g = gms_r[...]
        var = m2 - 2.0 * g * m1 * m1 + g * g * m1 * m1
        inv = lax.rsqrt(var + 1e-5)
        alpha = gw_r[...] * inv
        beta = gb_r[...] - gw_r[...] * g * m1 * inv
        z = act_r[...] * alpha + beta
        bmax = jnp.max(z, axis=0, keepdims=True)

        @pl.when(i == 0)
        def _():
            mx_s[...] = bmax

        @pl.when(i > 0)
        def _():
            mx_s[...] = jnp.maximum(mx_s[...], bmax)

        @pl.when(i == _G - 1)
        def _():
            out_r[...] = jnp.dot(mx_s[...], wfc_r[...],
                                 preferred_element_type=_f32) + bfc_r[...]

    full = lambda shape: pl.BlockSpec(shape, lambda i: tuple(0 for _ in shape))
    return pl.pallas_call(
        body,
        grid=(_G,),
        in_specs=[pl.BlockSpec((_BN, 32), lambda i: (i, 0)),
                  full((1, 32)), full((1, 32)), full((1, 32)), full((1, 32)),
                  full((1, 32)), full((32, 64)), full((1, 64))],
        out_specs=pl.BlockSpec((1, 64), lambda i: (0, 0)),
        out_shape=jax.ShapeDtypeStruct((1, 64), _f32),
        scratch_shapes=[pltpu.VMEM((1, 32), _f32)],
    )(act, pm1, pm2, gw, gb, gms, W_fc, b_fc)


def _gat_edge_phase(hp, a_s, a_d, mhat, src2d, dst2d):
    ha = hp[0]
    hb = hp[1]
    u2d = _p1_gather(src2d, a_s.reshape(_N))
    mhat16 = mhat.reshape(16)
    e2d, spart = _p2_softmax_num(dst2d, u2d, a_d.reshape(_N), mhat16)
    spart = spart.reshape(_NC, _N2)
    accp = _pb_aggregate(src2d, dst2d, e2d, ha, hb)
    return accp, spart


def kernel(x, edge_index, W_enc, b_enc, W1, a_src1, a_dst1, b1, g1_w, g1_b,
           g1_ms, W2, a_src2, a_dst2, b2, g2_w, g2_b, g2_ms, W_fc, b_fc):
    src = edge_index[0]
    dst = edge_index[1]
    padn = _E2 - _E
    src2d = jnp.concatenate([src, jnp.zeros((padn,), _i32)]).reshape(_R2, 128)
    dst2d = jnp.concatenate([dst, jnp.zeros((padn,), _i32)]).reshape(_R2, 128)

    r2 = lambda v: v.reshape(1, -1)

    h0 = _enc_call(x, W_enc, r2(b_enc))

    hp1, as1, ad1, mh1 = _prep_call(h0, W1, r2(a_src1), r2(a_dst1), None, 64)
    accp1, sp1 = _gat_edge_phase(hp1, as1, ad1, mh1, src2d, dst2d)
    act1, pm1, pm2 = _post_call(accp1[0], accp1[1],
                                sp1[0].reshape(_N2, 1), sp1[1].reshape(_N2, 1),
                                as1, ad1, mh1, hp1[0], hp1[1], r2(b1))

    hp2, as2, ad2, mh2 = _prep_call(
        act1, W2, r2(a_src2), r2(a_dst2),
        (pm1, pm2, r2(g1_w), r2(g1_b), r2(g1_ms)), 32)
    accp2, sp2 = _gat_edge_phase(hp2, as2, ad2, mh2, src2d, dst2d)
    act2, pm1b, pm2b = _post_call(accp2[0], accp2[1],
                                  sp2[0].reshape(_N2, 1), sp2[1].reshape(_N2, 1),
                                  as2, ad2, mh2, hp2[0], hp2[1], r2(b2))

    return _final_call(act2, pm1b, pm2b, r2(g2_w), r2(g2_b), r2(g2_ms),
                       W_fc, r2(b_fc))


# speedup vs baseline: 38.5770x; 1.2402x over previous
import functools

import jax
import jax.numpy as jnp
from jax import lax
from jax.experimental import pallas as pl
from jax.experimental.pallas import tpu as pltpu
from jax.experimental.pallas import tpu_sc as plsc

_N = 100000
_E = 1600000
_NC, _NS = 2, 16
_NW = _NC * _NS
_E2 = 1638400
_R2 = _E2 // 128
_RREAL = _E // 128
_RPW = _R2 // _NW
_RPT = _R2 // _NS
_N2 = 100096
_NPT = _N2 // _NS
_BN = 2000
_G = _N // _BN

_f32 = jnp.float32
_i32 = jnp.int32


def _sc_mesh():
    return plsc.VectorSubcoreMesh(core_axis_name="c", subcore_axis_name="s",
                                  num_cores=_NC, num_subcores=_NS)


def _p1_gather(src2d, tab):
    @functools.partial(
        pl.kernel,
        out_type=jax.ShapeDtypeStruct((_R2, 128), _f32),
        mesh=_sc_mesh(),
        compiler_params=pltpu.CompilerParams(needs_layout_passes=False, use_tc_tiling_on_sc=False),
        scratch_types=[
            pltpu.VMEM((_N,), _f32),
            pltpu.VMEM((16, 128), _i32),
            pltpu.VMEM((16, 128), _f32),
        ],
    )
    def k(src_h, tab_h, u_h, tab_v, idx_v, u_v):
        cid = lax.axis_index("c")
        sid = lax.axis_index("s")
        wid = sid * _NC + cid
        pltpu.sync_copy(tab_h, tab_v)
        rbase = wid * _RPW
        fill = jnp.full((16,), -1e30, _f32)

        def chunk(ci, _):
            roff = rbase + ci * 16
            pltpu.sync_copy(src_h.at[pl.ds(roff, 16)], idx_v)

            def grp(g, _):
                kk = g // 8
                ii = g % 8
                idx = idx_v[kk, pl.ds(ii * 16, 16)]
                vals = plsc.load_gather(tab_v, [idx])
                real = (roff + kk) < _RREAL
                u_v[kk, pl.ds(ii * 16, 16)] = jnp.where(real, vals, fill)
                return 0

            lax.fori_loop(0, 128, grp, 0)
            pltpu.sync_copy(u_v, u_h.at[pl.ds(roff, 16)])
            return 0

        lax.fori_loop(0, _RPW // 16, chunk, 0)

    return k(src2d, tab)


def _p2_softmax_num(dst2d, u2d, tab, mhat16):
    @functools.partial(
        pl.kernel,
        out_type=(
            jax.ShapeDtypeStruct((_R2, 128), _f32),
            jax.ShapeDtypeStruct((_NC * _N2,), _f32),
        ),
        mesh=_sc_mesh(),
        compiler_params=pltpu.CompilerParams(needs_layout_passes=False, use_tc_tiling_on_sc=False),
        scratch_types=[
            pltpu.VMEM((_N,), _f32),
            pltpu.VMEM((16, 128), _i32),
            pltpu.VMEM((16, 128), _f32),
            pltpu.VMEM((16, 128), _f32),
            pltpu.VMEM((16,), _f32),
            pltpu.VMEM((_NPT,), _f32),
            pltpu.VMEM_SHARED((_N2,), _f32),
        ],
    )
    def k(dst_h, u_h, tab_h, mh_h, e_h, sp_h,
          tab_v, didx_v, u_v, e_v, mh_v, zb_v, s_sh):
        cid = lax.axis_index("c")
        sid = lax.axis_index("s")
        wid = sid * _NC + cid
        pltpu.sync_copy(tab_h, tab_v)
        pltpu.sync_copy(mh_h, mh_v)

        def zi(i, _):
            zb_v[pl.ds(i * 16, 16)] = jnp.zeros((16,), _f32)
            return 0

        lax.fori_loop(0, _NPT // 16, zi, 0)
        pltpu.sync_copy(zb_v, s_sh.at[pl.ds(sid * _NPT, _NPT)])
        plsc.subcore_barrier()
        mh = mh_v[...]
        rbase = wid * _RPW

        def chunk(ci, _):
            roff = rbase + ci * 16
            pltpu.sync_copy(dst_h.at[pl.ds(roff, 16)], didx_v)
            pltpu.sync_copy(u_h.at[pl.ds(roff, 16)], u_v)

            def grp(g, _):
                kk = g // 8
                ii = g % 8
                idx = didx_v[kk, pl.ds(ii * 16, 16)]
                v = plsc.load_gather(tab_v, [idx])
                a = u_v[kk, pl.ds(ii * 16, 16)] + v
                a = jnp.where(a > 0, a, 0.2 * a)
                e_v[kk, pl.ds(ii * 16, 16)] = jnp.exp(a - mh)
                return 0

            lax.fori_loop(0, 128, grp, 0)
            pltpu.sync_copy(e_v, e_h.at[pl.ds(roff, 16)])

            def sca(kk, _):
                pltpu.sync_copy(e_v.at[kk], s_sh.at[didx_v.at[kk]], add=True)
                return 0

            lax.fori_loop(0, 16, sca, 0)
            return 0

        lax.fori_loop(0, _RPW // 16, chunk, 0)
        plsc.subcore_barrier()
        pltpu.sync_copy(s_sh.at[pl.ds(sid * _NPT, _NPT)], zb_v)
        pltpu.sync_copy(zb_v, sp_h.at[pl.ds(cid * _N2 + sid * _NPT, _NPT)])

    return k(dst2d, u2d, tab, mhat16)


def _pb_aggregate(src2d, dst2d, e2d, ha, hb):
    @functools.partial(
        pl.kernel,
        out_type=jax.ShapeDtypeStruct((_NC, _N2, 16), _f32),
        mesh=_sc_mesh(),
        compiler_params=pltpu.CompilerParams(needs_layout_passes=False, use_tc_tiling_on_sc=False),
        scratch_types=[
            pltpu.VMEM((16, 128), _i32),
            pltpu.VMEM((16, 128), _i32),
            pltpu.VMEM((16, 128), _f32),
            pltpu.VMEM((128, 16), _f32),
            pltpu.VMEM((128, 16), _f32),
            pltpu.VMEM((368, 16), _f32),
            pltpu.VMEM_SHARED((_N2, 16), _f32),
            pltpu.SemaphoreType.DMA,
            pltpu.SemaphoreType.DMA,
        ],
    )
    def k(src_h, dst_h, e_h, ha_h, hb_h, acc_h,
          sidx_v, didx_v, e_v, rows0_v, rows1_v, zb_v, acc_sh, gsem, ssem):
        cid = lax.axis_index("c")
        sid = lax.axis_index("s")

        def zi(g, _):
            zb_v[g] = jnp.zeros((16,), _f32)
            return 0

        lax.fori_loop(0, 368, zi, 0)

        def zc(p, _):
            pltpu.sync_copy(zb_v, acc_sh.at[pl.ds(sid * _NPT + p * 368, 368)])
            return 0

        lax.fori_loop(0, _NPT // 368, zc, 0)
        plsc.subcore_barrier()
        rbase = sid * _RPT

        def run(tab_h):
            rows = (rows0_v, rows1_v)

            def chunk(ci, _):
                roff = rbase + ci * 16
                pltpu.sync_copy(src_h.at[pl.ds(roff, 16)], sidx_v)
                pltpu.sync_copy(dst_h.at[pl.ds(roff, 16)], didx_v)
                pltpu.sync_copy(e_h.at[pl.ds(roff, 16)], e_v)
                gdesc = [None] * 16
                sdesc = [None] * 16
                gdesc[0] = pltpu.async_copy(tab_h.at[sidx_v.at[0]],
                                            rows[0], gsem)
                for kk in range(16):
                    if kk + 1 < 16:
                        if kk >= 1:
                            sdesc[kk - 1].wait()
                        gdesc[kk + 1] = pltpu.async_copy(
                            tab_h.at[sidx_v.at[kk + 1]],
                            rows[(kk + 1) % 2], gsem)
                    gdesc[kk].wait()
                    rv = rows[kk % 2]

                    def scl(g, _, kk=kk, rv=rv):
                        ev = e_v[kk, pl.ds(g * 16, 16)]
                        base = g * 16
                        for j in range(16):
                            rv[base + j] = rv[base + j] * ev[j]
                        return 0

                    lax.fori_loop(0, 8, scl, 0)
                    sdesc[kk] = pltpu.async_copy(
                        rv, acc_sh.at[didx_v.at[kk]], ssem, add=True)
                sdesc[14].wait()
                sdesc[15].wait()
                return 0

            lax.fori_loop(0, _RPT // 16, chunk, 0)

        @pl.when(cid == 0)
        def _():
            run(ha_h)

        @pl.when(cid == 1)
        def _():
            run(hb_h)

        plsc.subcore_barrier()

        def wc(p, _):
            off = sid * _NPT + p * 368
            pltpu.sync_copy(acc_sh.at[pl.ds(off, 368)], zb_v)
            pltpu.sync_copy(zb_v, acc_h.at[cid, pl.ds(off, 368)])
            return 0

        lax.fori_loop(0, _NPT // 368, wc, 0)

    return k(src2d, dst2d, e2d, ha, hb)


def _prep_call(act, W, a_src, a_dst, fold_params, din):
    folded = fold_params is not None

    def body(*refs):
        if folded:
            (act_r, pm1_r, pm2_r, gw_r, gb_r, gms_r, w_r, asr_r, adr_r,
             hp_r, as_r, ad_r, mh_r, ms_s, md_s) = refs
        else:
            (act_r, w_r, asr_r, adr_r,
             hp_r, as_r, ad_r, mh_r, ms_s, md_s) = refs
        i = pl.program_id(0)
        z = act_r[...]
        if folded:
            m1 = pm1_r[...] / _N
            m2 = pm2_r[...] / _N
            gms = gms_r[...]
            var = m2 - 2.0 * gms * m1 * m1 + gms * gms * m1 * m1
            inv = lax.rsqrt(var + 1e-5)
            alpha = gw_r[...] * inv
            beta = gb_r[...] - gw_r[...] * gms * m1 * inv
            z = z * alpha + beta
        h = jnp.dot(z, w_r[...], preferred_element_type=_f32)
        hp_r[0] = h[:, :16]
        hp_r[1] = h[:, 16:]
        a_s = jnp.sum(h * asr_r[...], axis=1, keepdims=True)
        a_d = jnp.sum(h * adr_r[...], axis=1, keepdims=True)
        as_r[...] = a_s
        ad_r[...] = a_d
        bm_s = jnp.max(a_s)
        bm_d = jnp.max(a_d)

        @pl.when(i == 0)
        def _():
            ms_s[0, 0] = bm_s
            md_s[0, 0] = bm_d

        @pl.when(i > 0)
        def _():
            ms_s[0, 0] = jnp.maximum(ms_s[0, 0], bm_s)
            md_s[0, 0] = jnp.maximum(md_s[0, 0], bm_d)

        @pl.when(i == _G - 1)
        def _():
            m = ms_s[0, 0] + md_s[0, 0]
            mh_r[...] = jnp.full((1, 16), jnp.where(m > 0, m, 0.2 * m), _f32)

    full = lambda shape: pl.BlockSpec(shape, lambda i: tuple(0 for _ in shape))
    in_specs = [pl.BlockSpec((_BN, din), lambda i: (i, 0))]
    args = [act]
    if folded:
        pm1, pm2, gw, gb, gms = fold_params
        in_specs += [full((1, 32))] * 5
        args += [pm1, pm2, gw, gb, gms]
    in_specs += [full((din, 32)), full((1, 32)), full((1, 32))]
    args += [W, a_src, a_dst]

    return pl.pallas_call(
        body,
        grid=(_G,),
        in_specs=in_specs,
        out_specs=[
            pl.BlockSpec((2, _BN, 16), lambda i: (0, i, 0)),
            pl.BlockSpec((_BN, 1), lambda i: (i, 0)),
            pl.BlockSpec((_BN, 1), lambda i: (i, 0)),
            pl.BlockSpec((1, 16), lambda i: (0, 0)),
        ],
        out_shape=[
            jax.ShapeDtypeStruct((2, _N, 16), _f32),
            jax.ShapeDtypeStruct((_N, 1), _f32),
            jax.ShapeDtypeStruct((_N, 1), _f32),
            jax.ShapeDtypeStruct((1, 16), _f32),
        ],
        scratch_shapes=[pltpu.SMEM((1, 1), _f32), pltpu.SMEM((1, 1), _f32)],
    )(*args)


def _enc_call(x, W_enc, b_enc):
    def body(x_r, we_r, be_r, h0_r):
        h0_r[...] = jnp.dot(x_r[...], we_r[...],
                            preferred_element_type=_f32) + be_r[...]

    return pl.pallas_call(
        body,
        grid=(_G,),
        in_specs=[
            pl.BlockSpec((_BN, 128), lambda i: (i, 0)),
            pl.BlockSpec((128, 64), lambda i: (0, 0)),
            pl.BlockSpec((1, 64), lambda i: (0, 0)),
        ],
        out_specs=pl.BlockSpec((_BN, 64), lambda i: (i, 0)),
        out_shape=jax.ShapeDtypeStruct((_N, 64), _f32),
    )(x, W_enc, b_enc)


def _post_call(acc0, acc1, s0, s1, a_s, a_d, mhat, ha, hb, b):
    def body(acc0_r, acc1_r, s0_r, s1_r, as_r, ad_r, mh_r, ha_r, hb_r, b_r,
             act_r, pm1_r, pm2_r):
        i = pl.program_id(0)
        mh0 = mh_r[0, 0]
        araw = as_r[...] + ad_r[...]
        alr = jnp.where(araw > 0, araw, 0.2 * araw)
        es = jnp.exp(alr - mh0)
        s = s0_r[...] + s1_r[...] + es
        acc = jnp.concatenate([acc0_r[...], acc1_r[...]], axis=1)
        hf = jnp.concatenate([ha_r[...], hb_r[...]], axis=1)
        acc = acc + es * hf
        o = acc / (s + 1e-16) + b_r[...]
        act = jnp.where(o > 0, o, jnp.exp(o) - 1.0)
        act_r[...] = act
        ps1 = jnp.sum(act, axis=0, keepdims=True)
        ps2 = jnp.sum(act * act, axis=0, keepdims=True)

        @pl.when(i == 0)
        def _():
            pm1_r[...] = ps1
            pm2_r[...] = ps2

        @pl.when(i > 0)
        def _():
            pm1_r[...] = pm1_r[...] + ps1
            pm2_r[...] = pm2_r[...] + ps2

    blk = lambda w: pl.BlockSpec((_BN, w), lambda i: (i, 0))
    full = lambda shape: pl.BlockSpec(shape, lambda i: tuple(0 for _ in shape))
    return pl.pallas_call(
        body,
        grid=(_G,),
        in_specs=[blk(16), blk(16), blk(1), blk(1), blk(1), blk(1),
                  full((1, 16)), blk(16), blk(16), full((1, 32))],
        out_specs=[
            pl.BlockSpec((_BN, 32), lambda i: (i, 0)),
            pl.BlockSpec((1, 32), lambda i: (0, 0)),
            pl.BlockSpec((1, 32), lambda i: (0, 0)),
        ],
        out_shape=[
            jax.ShapeDtypeStruct((_N, 32), _f32),
            jax.ShapeDtypeStruct((1, 32), _f32),
            jax.ShapeDtypeStruct((1, 32), _f32),
        ],
    )(acc0, acc1, s0, s1, a_s, a_d, mhat, ha, hb, b)


def _final_call(act, pm1, pm2, gw, gb, gms, W_fc, b_fc):
    def body(act_r, pm1_r, pm2_r, gw_r, gb_r, gms_r, wfc_r, bfc_r,
             out_r, mx_s):
        i = pl.program_id(0)
        m1 = pm1_r[...] / _N
        m2 = pm2_r[...] / _N
        g = gms_r[...]
        var = m2 - 2.0 * g * m1 * m1 + g * g * m1 * m1
        inv = lax.rsqrt(var + 1e-5)
        alpha = gw_r[...] * inv
        beta = gb_r[...] - gw_r[...] * g * m1 * inv
        z = act_r[...] * alpha + beta
        bmax = jnp.max(z, axis=0, keepdims=True)

        @pl.when(i == 0)
        def _():
            mx_s[...] = bmax

        @pl.when(i > 0)
        def _():
            mx_s[...] = jnp.maximum(mx_s[...], bmax)

        @pl.when(i == _G - 1)
        def _():
            out_r[...] = jnp.dot(mx_s[...], wfc_r[...],
                                 preferred_element_type=_f32) + bfc_r[...]

    full = lambda shape: pl.BlockSpec(shape, lambda i: tuple(0 for _ in shape))
    return pl.pallas_call(
        body,
        grid=(_G,),
        in_specs=[pl.BlockSpec((_BN, 32), lambda i: (i, 0)),
                  full((1, 32)), full((1, 32)), full((1, 32)), full((1, 32)),
                  full((1, 32)), full((32, 64)), full((1, 64))],
        out_specs=pl.BlockSpec((1, 64), lambda i: (0, 0)),
        out_shape=jax.ShapeDtypeStruct((1, 64), _f32),
        scratch_shapes=[pltpu.VMEM((1, 32), _f32)],
    )(act, pm1, pm2, gw, gb, gms, W_fc, b_fc)


def _gat_edge_phase(hp, a_s, a_d, mhat, src2d, dst2d):
    ha = hp[0]
    hb = hp[1]
    u2d = _p1_gather(src2d, a_s.reshape(_N))
    mhat16 = mhat.reshape(16)
    e2d, spart = _p2_softmax_num(dst2d, u2d, a_d.reshape(_N), mhat16)
    spart = spart.reshape(_NC, _N2)
    accp = _pb_aggregate(src2d, dst2d, e2d, ha, hb)
    return accp, spart


def kernel(x, edge_index, W_enc, b_enc, W1, a_src1, a_dst1, b1, g1_w, g1_b,
           g1_ms, W2, a_src2, a_dst2, b2, g2_w, g2_b, g2_ms, W_fc, b_fc):
    src = edge_index[0]
    dst = edge_index[1]
    padn = _E2 - _E
    src2d = jnp.concatenate([src, jnp.zeros((padn,), _i32)]).reshape(_R2, 128)
    dst2d = jnp.concatenate([dst, jnp.zeros((padn,), _i32)]).reshape(_R2, 128)

    r2 = lambda v: v.reshape(1, -1)

    h0 = _enc_call(x, W_enc, r2(b_enc))

    hp1, as1, ad1, mh1 = _prep_call(h0, W1, r2(a_src1), r2(a_dst1), None, 64)
    accp1, sp1 = _gat_edge_phase(hp1, as1, ad1, mh1, src2d, dst2d)
    act1, pm1, pm2 = _post_call(accp1[0], accp1[1],
                                sp1[0].reshape(_N2, 1), sp1[1].reshape(_N2, 1),
                                as1, ad1, mh1, hp1[0], hp1[1], r2(b1))

    hp2, as2, ad2, mh2 = _prep_call(
        act1, W2, r2(a_src2), r2(a_dst2),
        (pm1, pm2, r2(g1_w), r2(g1_b), r2(g1_ms)), 32)
    accp2, sp2 = _gat_edge_phase(hp2, as2, ad2, mh2, src2d, dst2d)
    act2, pm1b, pm2b = _post_call(accp2[0], accp2[1],
                                  sp2[0].reshape(_N2, 1), sp2[1].reshape(_N2, 1),
                                  as2, ad2, mh2, hp2[0], hp2[1], r2(b2))

    return _final_call(act2, pm1b, pm2b, r2(g2_w), r2(g2_b), r2(g2_ms),
                       W_fc, r2(b_fc))


# speedup vs baseline: 42.0665x; 1.0905x over previous
import functools

import jax
import jax.numpy as jnp
from jax import lax
from jax.experimental import pallas as pl
from jax.experimental.pallas import tpu as pltpu
from jax.experimental.pallas import tpu_sc as plsc

_N = 100000
_E = 1600000
_NC, _NS = 2, 16
_NW = _NC * _NS
_E2 = 1638400
_R2 = _E2 // 128
_RREAL = _E // 128
_RPW = _R2 // _NW
_RPT = _R2 // _NS
_N2 = 100096
_NPT = _N2 // _NS
_BN = 2000
_G = _N // _BN

_f32 = jnp.float32
_i32 = jnp.int32


def _sc_mesh():
    return plsc.VectorSubcoreMesh(core_axis_name="c", subcore_axis_name="s",
                                  num_cores=_NC, num_subcores=_NS)


def _p1_gather(src2d, tab):
    @functools.partial(
        pl.kernel,
        out_type=jax.ShapeDtypeStruct((_R2, 128), _f32),
        mesh=_sc_mesh(),
        compiler_params=pltpu.CompilerParams(needs_layout_passes=False, use_tc_tiling_on_sc=False),
        scratch_types=[
            pltpu.VMEM((_N,), _f32),
            pltpu.VMEM((16, 128), _i32),
            pltpu.VMEM((16, 128), _f32),
        ],
    )
    def k(src_h, tab_h, u_h, tab_v, idx_v, u_v):
        cid = lax.axis_index("c")
        sid = lax.axis_index("s")
        wid = sid * _NC + cid
        pltpu.sync_copy(tab_h, tab_v)
        rbase = wid * _RPW
        fill = jnp.full((16,), -1e30, _f32)

        def chunk(ci, _):
            roff = rbase + ci * 16
            pltpu.sync_copy(src_h.at[pl.ds(roff, 16)], idx_v)

            def grp(g, _):
                kk = g // 8
                ii = g % 8
                idx = idx_v[kk, pl.ds(ii * 16, 16)]
                vals = plsc.load_gather(tab_v, [idx])
                real = (roff + kk) < _RREAL
                u_v[kk, pl.ds(ii * 16, 16)] = jnp.where(real, vals, fill)
                return 0

            lax.fori_loop(0, 128, grp, 0)
            pltpu.sync_copy(u_v, u_h.at[pl.ds(roff, 16)])
            return 0

        lax.fori_loop(0, _RPW // 16, chunk, 0)

    return k(src2d, tab)


def _p2_softmax_num(dst2d, u2d, tab, mhat16):
    @functools.partial(
        pl.kernel,
        out_type=(
            jax.ShapeDtypeStruct((_R2, 128), _f32),
            jax.ShapeDtypeStruct((_NC * _N2,), _f32),
        ),
        mesh=_sc_mesh(),
        compiler_params=pltpu.CompilerParams(needs_layout_passes=False, use_tc_tiling_on_sc=False),
        scratch_types=[
            pltpu.VMEM((_N,), _f32),
            pltpu.VMEM((16, 128), _i32),
            pltpu.VMEM((16, 128), _f32),
            pltpu.VMEM((16, 128), _f32),
            pltpu.VMEM((16,), _f32),
            pltpu.VMEM((_NPT,), _f32),
            pltpu.VMEM_SHARED((_N2,), _f32),
            pltpu.SemaphoreType.DMA,
            pltpu.SemaphoreType.DMA,
        ],
    )
    def k(dst_h, u_h, tab_h, mh_h, e_h, sp_h,
          tab_v, didx_v, u_v, e_v, mh_v, zb_v, s_sh, ssem, osem):
        cid = lax.axis_index("c")
        sid = lax.axis_index("s")
        wid = sid * _NC + cid
        pltpu.sync_copy(tab_h, tab_v)
        pltpu.sync_copy(mh_h, mh_v)

        def zi(i, _):
            zb_v[pl.ds(i * 16, 16)] = jnp.zeros((16,), _f32)
            return 0

        lax.fori_loop(0, _NPT // 16, zi, 0)
        pltpu.sync_copy(zb_v, s_sh.at[pl.ds(sid * _NPT, _NPT)])
        plsc.subcore_barrier()
        mh = mh_v[...]
        rbase = wid * _RPW

        def chunk(ci, _):
            roff = rbase + ci * 16
            pltpu.sync_copy(dst_h.at[pl.ds(roff, 16)], didx_v)
            pltpu.sync_copy(u_h.at[pl.ds(roff, 16)], u_v)

            def grp(g, _):
                kk = g // 8
                ii = g % 8
                idx = didx_v[kk, pl.ds(ii * 16, 16)]
                v = plsc.load_gather(tab_v, [idx])
                a = u_v[kk, pl.ds(ii * 16, 16)] + v
                a = jnp.where(a > 0, a, 0.2 * a)
                e_v[kk, pl.ds(ii * 16, 16)] = jnp.exp(a - mh)
                return 0

            lax.fori_loop(0, 128, grp, 0)
            odesc = pltpu.async_copy(e_v, e_h.at[pl.ds(roff, 16)], osem)
            sdesc = [pltpu.async_copy(e_v.at[kk], s_sh.at[didx_v.at[kk]],
                                      ssem, add=True) for kk in range(16)]
            for d in sdesc:
                d.wait()
            odesc.wait()
            return 0

        lax.fori_loop(0, _RPW // 16, chunk, 0)
        plsc.subcore_barrier()
        pltpu.sync_copy(s_sh.at[pl.ds(sid * _NPT, _NPT)], zb_v)
        pltpu.sync_copy(zb_v, sp_h.at[pl.ds(cid * _N2 + sid * _NPT, _NPT)])

    return k(dst2d, u2d, tab, mhat16)


def _pb_aggregate(src2d, dst2d, e2d, ha, hb):
    @functools.partial(
        pl.kernel,
        out_type=jax.ShapeDtypeStruct((_NC, _N2, 16), _f32),
        mesh=_sc_mesh(),
        compiler_params=pltpu.CompilerParams(needs_layout_passes=False, use_tc_tiling_on_sc=False),
        scratch_types=[
            pltpu.VMEM((16, 128), _i32),
            pltpu.VMEM((16, 128), _i32),
            pltpu.VMEM((16, 128), _f32),
            pltpu.VMEM((128, 16), _f32),
            pltpu.VMEM((128, 16), _f32),
            pltpu.VMEM((128, 16), _f32),
            pltpu.VMEM((128, 16), _f32),
            pltpu.VMEM((368, 16), _f32),
            pltpu.VMEM_SHARED((_N2, 16), _f32),
            pltpu.SemaphoreType.DMA,
            pltpu.SemaphoreType.DMA,
        ],
    )
    def k(src_h, dst_h, e_h, ha_h, hb_h, acc_h,
          sidx_v, didx_v, e_v, rows0_v, rows1_v, rows2_v, rows3_v,
          zb_v, acc_sh, gsem, ssem):
        cid = lax.axis_index("c")
        sid = lax.axis_index("s")

        def zi(g, _):
            zb_v[g] = jnp.zeros((16,), _f32)
            return 0

        lax.fori_loop(0, 368, zi, 0)

        def zc(p, _):
            pltpu.sync_copy(zb_v, acc_sh.at[pl.ds(sid * _NPT + p * 368, 368)])
            return 0

        lax.fori_loop(0, _NPT // 368, zc, 0)
        plsc.subcore_barrier()
        rbase = sid * _RPT

        def run(tab_h):
            rows = (rows0_v, rows1_v, rows2_v, rows3_v)

            def chunk(ci, _):
                roff = rbase + ci * 16
                pltpu.sync_copy(src_h.at[pl.ds(roff, 16)], sidx_v)
                pltpu.sync_copy(dst_h.at[pl.ds(roff, 16)], didx_v)
                pltpu.sync_copy(e_h.at[pl.ds(roff, 16)], e_v)
                gdesc = [None] * 16
                sdesc = [None] * 16
                for b in range(3):
                    gdesc[b] = pltpu.async_copy(tab_h.at[sidx_v.at[b]],
                                                rows[b], gsem)
                for kk in range(16):
                    if kk + 3 < 16:
                        if kk >= 1:
                            sdesc[kk - 1].wait()
                        gdesc[kk + 3] = pltpu.async_copy(
                            tab_h.at[sidx_v.at[kk + 3]],
                            rows[(kk + 3) % 4], gsem)
                    gdesc[kk].wait()
                    rv = rows[kk % 4]

                    def scl(g, _, kk=kk, rv=rv):
                        ev = e_v[kk, pl.ds(g * 16, 16)]
                        base = g * 16
                        for j in range(16):
                            rv[base + j] = rv[base + j] * ev[j]
                        return 0

                    lax.fori_loop(0, 8, scl, 0)
                    sdesc[kk] = pltpu.async_copy(
                        rv, acc_sh.at[didx_v.at[kk]], ssem, add=True)
                for kk in range(12, 16):
                    sdesc[kk].wait()
                return 0

            lax.fori_loop(0, _RPT // 16, chunk, 0)

        @pl.when(cid == 0)
        def _():
            run(ha_h)

        @pl.when(cid == 1)
        def _():
            run(hb_h)

        plsc.subcore_barrier()

        def wc(p, _):
            off = sid * _NPT + p * 368
            pltpu.sync_copy(acc_sh.at[pl.ds(off, 368)], zb_v)
            pltpu.sync_copy(zb_v, acc_h.at[cid, pl.ds(off, 368)])
            return 0

        lax.fori_loop(0, _NPT // 368, wc, 0)

    return k(src2d, dst2d, e2d, ha, hb)


def _prep_call(act, W, a_src, a_dst, fold_params, din):
    folded = fold_params is not None

    def body(*refs):
        if folded:
            (act_r, pm1_r, pm2_r, gw_r, gb_r, gms_r, w_r, asr_r, adr_r,
             hp_r, as_r, ad_r, mh_r, ms_s, md_s) = refs
        else:
            (act_r, w_r, asr_r, adr_r,
             hp_r, as_r, ad_r, mh_r, ms_s, md_s) = refs
        i = pl.program_id(0)
        z = act_r[...]
        if folded:
            m1 = pm1_r[...] / _N
            m2 = pm2_r[...] / _N
            gms = gms_r[...]
            var = m2 - 2.0 * gms * m1 * m1 + gms * gms * m1 * m1
            inv = lax.rsqrt(var + 1e-5)
            alpha = gw_r[...] * inv
            beta = gb_r[...] - gw_r[...] * gms * m1 * inv
            z = z * alpha + beta
        h = jnp.dot(z, w_r[...], preferred_element_type=_f32)
        hp_r[0] = h[:, :16]
        hp_r[1] = h[:, 16:]
        a_s = jnp.sum(h * asr_r[...], axis=1, keepdims=True)
        a_d = jnp.sum(h * adr_r[...], axis=1, keepdims=True)
        as_r[...] = a_s
        ad_r[...] = a_d
        bm_s = jnp.max(a_s)
        bm_d = jnp.max(a_d)

        @pl.when(i == 0)
        def _():
            ms_s[0, 0] = bm_s
            md_s[0, 0] = bm_d

        @pl.when(i > 0)
        def _():
            ms_s[0, 0] = jnp.maximum(ms_s[0, 0], bm_s)
            md_s[0, 0] = jnp.maximum(md_s[0, 0], bm_d)

        @pl.when(i == _G - 1)
        def _():
            m = ms_s[0, 0] + md_s[0, 0]
            mh_r[...] = jnp.full((1, 16), jnp.where(m > 0, m, 0.2 * m), _f32)

    full = lambda shape: pl.BlockSpec(shape, lambda i: tuple(0 for _ in shape))
    in_specs = [pl.BlockSpec((_BN, din), lambda i: (i, 0))]
    args = [act]
    if folded:
        pm1, pm2, gw, gb, gms = fold_params
        in_specs += [full((1, 32))] * 5
        args += [pm1, pm2, gw, gb, gms]
    in_specs += [full((din, 32)), full((1, 32)), full((1, 32))]
    args += [W, a_src, a_dst]

    return pl.pallas_call(
        body,
        grid=(_G,),
        in_specs=in_specs,
        out_specs=[
            pl.BlockSpec((2, _BN, 16), lambda i: (0, i, 0)),
            pl.BlockSpec((_BN, 1), lambda i: (i, 0)),
            pl.BlockSpec((_BN, 1), lambda i: (i, 0)),
            pl.BlockSpec((1, 16), lambda i: (0, 0)),
        ],
        out_shape=[
            jax.ShapeDtypeStruct((2, _N, 16), _f32),
            jax.ShapeDtypeStruct((_N, 1), _f32),
            jax.ShapeDtypeStruct((_N, 1), _f32),
            jax.ShapeDtypeStruct((1, 16), _f32),
        ],
        scratch_shapes=[pltpu.SMEM((1, 1), _f32), pltpu.SMEM((1, 1), _f32)],
    )(*args)


def _enc_call(x, W_enc, b_enc):
    def body(x_r, we_r, be_r, h0_r):
        h0_r[...] = jnp.dot(x_r[...], we_r[...],
                            preferred_element_type=_f32) + be_r[...]

    return pl.pallas_call(
        body,
        grid=(_G,),
        in_specs=[
            pl.BlockSpec((_BN, 128), lambda i: (i, 0)),
            pl.BlockSpec((128, 64), lambda i: (0, 0)),
            pl.BlockSpec((1, 64), lambda i: (0, 0)),
        ],
        out_specs=pl.BlockSpec((_BN, 64), lambda i: (i, 0)),
        out_shape=jax.ShapeDtypeStruct((_N, 64), _f32),
    )(x, W_enc, b_enc)


def _post_call(acc0, acc1, s0, s1, a_s, a_d, mhat, ha, hb, b):
    def body(acc0_r, acc1_r, s0_r, s1_r, as_r, ad_r, mh_r, ha_r, hb_r, b_r,
             act_r, pm1_r, pm2_r):
        i = pl.program_id(0)
        mh0 = mh_r[0, 0]
        araw = as_r[...] + ad_r[...]
        alr = jnp.where(araw > 0, araw, 0.2 * araw)
        es = jnp.exp(alr - mh0)
        s = s0_r[...] + s1_r[...] + es
        acc = jnp.concatenate([acc0_r[...], acc1_r[...]], axis=1)
        hf = jnp.concatenate([ha_r[...], hb_r[...]], axis=1)
        acc = acc + es * hf
        o = acc / (s + 1e-16) + b_r[...]
        act = jnp.where(o > 0, o, jnp.exp(o) - 1.0)
        act_r[...] = act
        ps1 = jnp.sum(act, axis=0, keepdims=True)
        ps2 = jnp.sum(act * act, axis=0, keepdims=True)

        @pl.when(i == 0)
        def _():
            pm1_r[...] = ps1
            pm2_r[...] = ps2

        @pl.when(i > 0)
        def _():
            pm1_r[...] = pm1_r[...] + ps1
            pm2_r[...] = pm2_r[...] + ps2

    blk = lambda w: pl.BlockSpec((_BN, w), lambda i: (i, 0))
    full = lambda shape: pl.BlockSpec(shape, lambda i: tuple(0 for _ in shape))
    return pl.pallas_call(
        body,
        grid=(_G,),
        in_specs=[blk(16), blk(16), blk(1), blk(1), blk(1), blk(1),
                  full((1, 16)), blk(16), blk(16), full((1, 32))],
        out_specs=[
            pl.BlockSpec((_BN, 32), lambda i: (i, 0)),
            pl.BlockSpec((1, 32), lambda i: (0, 0)),
            pl.BlockSpec((1, 32), lambda i: (0, 0)),
        ],
        out_shape=[
            jax.ShapeDtypeStruct((_N, 32), _f32),
            jax.ShapeDtypeStruct((1, 32), _f32),
            jax.ShapeDtypeStruct((1, 32), _f32),
        ],
    )(acc0, acc1, s0, s1, a_s, a_d, mhat, ha, hb, b)


def _final_call(act, pm1, pm2, gw, gb, gms, W_fc, b_fc):
    def body(act_r, pm1_r, pm2_r, gw_r, gb_r, gms_r, wfc_r, bfc_r,
             out_r, mx_s):
        i = pl.program_id(0)
        m1 = pm1_r[...] / _N
        m2 = pm2_r[...] / _N
        g = gms_r[...]
        var = m2 - 2.0 * g * m1 * m1 + g * g * m1 * m1
        inv = lax.rsqrt(var + 1e-5)
        alpha = gw_r[...] * inv
        beta = gb_r[...] - gw_r[...] * g * m1 * inv
        z = act_r[...] * alpha + beta
        bmax = jnp.max(z, axis=0, keepdims=True)

        @pl.when(i == 0)
        def _():
            mx_s[...] = bmax

        @pl.when(i > 0)
        def _():
            mx_s[...] = jnp.maximum(mx_s[...], bmax)

        @pl.when(i == _G - 1)
        def _():
            out_r[...] = jnp.dot(mx_s[...], wfc_r[...],
                                 preferred_element_type=_f32) + bfc_r[...]

    full = lambda shape: pl.BlockSpec(shape, lambda i: tuple(0 for _ in shape))
    return pl.pallas_call(
        body,
        grid=(_G,),
        in_specs=[pl.BlockSpec((_BN, 32), lambda i: (i, 0)),
                  full((1, 32)), full((1, 32)), full((1, 32)), full((1, 32)),
                  full((1, 32)), full((32, 64)), full((1, 64))],
        out_specs=pl.BlockSpec((1, 64), lambda i: (0, 0)),
        out_shape=jax.ShapeDtypeStruct((1, 64), _f32),
        scratch_shapes=[pltpu.VMEM((1, 32), _f32)],
    )(act, pm1, pm2, gw, gb, gms, W_fc, b_fc)


def _gat_edge_phase(hp, a_s, a_d, mhat, src2d, dst2d):
    ha = hp[0]
    hb = hp[1]
    u2d = _p1_gather(src2d, a_s.reshape(_N))
    mhat16 = mhat.reshape(16)
    e2d, spart = _p2_softmax_num(dst2d, u2d, a_d.reshape(_N), mhat16)
    spart = spart.reshape(_NC, _N2)
    accp = _pb_aggregate(src2d, dst2d, e2d, ha, hb)
    return accp, spart


def kernel(x, edge_index, W_enc, b_enc, W1, a_src1, a_dst1, b1, g1_w, g1_b,
           g1_ms, W2, a_src2, a_dst2, b2, g2_w, g2_b, g2_ms, W_fc, b_fc):
    src = edge_index[0]
    dst = edge_index[1]
    padn = _E2 - _E
    src2d = jnp.concatenate([src, jnp.zeros((padn,), _i32)]).reshape(_R2, 128)
    dst2d = jnp.concatenate([dst, jnp.zeros((padn,), _i32)]).reshape(_R2, 128)

    r2 = lambda v: v.reshape(1, -1)

    h0 = _enc_call(x, W_enc, r2(b_enc))

    hp1, as1, ad1, mh1 = _prep_call(h0, W1, r2(a_src1), r2(a_dst1), None, 64)
    accp1, sp1 = _gat_edge_phase(hp1, as1, ad1, mh1, src2d, dst2d)
    act1, pm1, pm2 = _post_call(accp1[0], accp1[1],
                                sp1[0].reshape(_N2, 1), sp1[1].reshape(_N2, 1),
                                as1, ad1, mh1, hp1[0], hp1[1], r2(b1))

    hp2, as2, ad2, mh2 = _prep_call(
        act1, W2, r2(a_src2), r2(a_dst2),
        (pm1, pm2, r2(g1_w), r2(g1_b), r2(g1_ms)), 32)
    accp2, sp2 = _gat_edge_phase(hp2, as2, ad2, mh2, src2d, dst2d)
    act2, pm1b, pm2b = _post_call(accp2[0], accp2[1],
                                  sp2[0].reshape(_N2, 1), sp2[1].reshape(_N2, 1),
                                  as2, ad2, mh2, hp2[0], hp2[1], r2(b2))

    return _final_call(act2, pm1b, pm2b, r2(g2_w), r2(g2_b), r2(g2_ms),
                       W_fc, r2(b_fc))
